# Initial kernel scaffold; baseline (speedup 1.0000x reference)
import jax
import jax.numpy as jnp
from jax.experimental import pallas as pl


def kernel(x, edge_index_0, edge_index_1, edge_index_2, W1_0, b1_0, W2_0, b2_0, W1_1, b1_1, W2_1, b2_1, W1_2, b1_2, W2_2, b2_2):
    raise NotImplementedError("write your pallas kernel here")


# speedup vs baseline: 1.6815x; 1.6815x over previous
import functools

import jax
import jax.numpy as jnp
from jax import lax
from jax.experimental import pallas as pl
from jax.experimental.pallas import tpu as pltpu
from jax.experimental.pallas import tpu_sc as plsc

N = 10000
E = 160000
D = 128
R = 3

NC = 2
NS = 16
NW = NC * NS

CHUNK = 128
E_PAD = 163840
EPW = E_PAD // NW
NCHUNK = EPW // CHUNK
ACC_ROWS = 10240
ROWS_PT = ACC_ROWS // NS
DEG_BINS = ACC_ROWS
NB = (N + 127) // 128

_mesh = lambda: plsc.VectorSubcoreMesh(core_axis_name="c", subcore_axis_name="s")


DEG_W = 128


def _deg_body(i0, i1, i2, i3, i4, i5, out_hbm,
              idx_v, ones_v, zeros_v, deg_sh):
    c = lax.axis_index("c")
    s = lax.axis_index("s")
    z16 = jnp.zeros((16,), jnp.float32)
    o16 = jnp.ones((16,), jnp.float32)

    def ob(t, carry):
        ones_v[t // 8, pl.ds((t % 8) * 16, 16)] = o16
        return carry
    lax.fori_loop(0, CHUNK * (DEG_W // 16), ob, 0)

    def zb(t, carry):
        zeros_v[t // 8, pl.ds((t % 8) * 16, 16)] = z16
        return carry
    lax.fori_loop(0, CHUNK * (DEG_W // 16), zb, 0)

    base = (c * NS + s) * EPW
    for h, idx_hbm in enumerate((i0, i1, i2, i3, i4, i5)):
        for k in range(ROWS_PT // CHUNK):
            pltpu.sync_copy(zeros_v,
                            deg_sh.at[pl.ds(s * ROWS_PT + k * CHUNK, CHUNK)])
        plsc.subcore_barrier()

        def cb(i, carry):
            pltpu.sync_copy(idx_hbm.at[pl.ds(base + i * CHUNK, CHUNK)], idx_v)
            pltpu.sync_copy(ones_v, deg_sh.at[idx_v], add=True)
            return carry
        lax.fori_loop(0, NCHUNK, cb, 0)
        plsc.subcore_barrier()

        pltpu.sync_copy(deg_sh.at[pl.ds(s * ROWS_PT, ROWS_PT)],
                        out_hbm.at[h, c, pl.ds(s * ROWS_PT, ROWS_PT)])
        plsc.subcore_barrier()


def _deg_call(deg_idx):
    return pl.kernel(
        _deg_body,
        out_type=jax.ShapeDtypeStruct((2 * R, NC, DEG_BINS, DEG_W), jnp.float32),
        mesh=_mesh(),
        scratch_types=[
            pltpu.VMEM((CHUNK,), jnp.int32),
            pltpu.VMEM((CHUNK, DEG_W), jnp.float32),
            pltpu.VMEM((CHUNK, DEG_W), jnp.float32),
            pltpu.VMEM_SHARED((DEG_BINS, DEG_W), jnp.float32),
        ],
    )(*deg_idx)


def _seg_body(h_hbm, s0, s1, s2, d0, d1, d2, o0, o1, o2,
              sv, dv, rows_v, zeros_v, acc, sem):
    c = lax.axis_index("c")
    s = lax.axis_index("s")
    z16 = jnp.zeros((16,), jnp.float32)

    def zb(t, carry):
        zeros_v[t // 8, pl.ds((t % 8) * 16, 16)] = z16
        return carry
    lax.fori_loop(0, CHUNK * (D // 16), zb, 0)

    srcs = (s0, s1, s2)
    dsts = (d0, d1, d2)
    outs = (o0, o1, o2)
    base = (c * NS + s) * EPW
    for r in range(R):
        for k in range(ROWS_PT // CHUNK):
            pltpu.sync_copy(zeros_v, acc.at[pl.ds(s * ROWS_PT + k * CHUNK, CHUNK)])
        plsc.subcore_barrier()

        def cb(i, carry):
            off = base + i * CHUNK
            pltpu.sync_copy(srcs[r].at[pl.ds(off, CHUNK)], sv)
            pltpu.sync_copy(dsts[r].at[pl.ds(off, CHUNK)], dv)
            pltpu.async_copy(h_hbm.at[r].at[sv], rows_v, sem).wait()
            pltpu.sync_copy(rows_v, acc.at[dv], add=True)
            return carry
        lax.fori_loop(0, NCHUNK, cb, 0)
        plsc.subcore_barrier()

        pltpu.sync_copy(acc.at[pl.ds(s * ROWS_PT, ROWS_PT)],
                        outs[r].at[c, pl.ds(s * ROWS_PT, ROWS_PT)])
        plsc.subcore_barrier()


def _seg_call(h3, src3, dst3):
    return pl.kernel(
        _seg_body,
        out_type=(jax.ShapeDtypeStruct((NC, ACC_ROWS, D), jnp.float32),) * R,
        mesh=_mesh(),
        scratch_types=[
            pltpu.VMEM((CHUNK,), jnp.int32),
            pltpu.VMEM((CHUNK,), jnp.int32),
            pltpu.VMEM((CHUNK, D), jnp.float32),
            pltpu.VMEM((CHUNK, D), jnp.float32),
            pltpu.VMEM_SHARED((ACC_ROWS, D), jnp.float32),
            pltpu.SemaphoreType.DMA,
        ],
    )(h3, *src3, *dst3)


NORM_BLK = 1024


def _norm_body(degp_ref, out_ref):
    e0 = jnp.where(lax.iota(jnp.int32, DEG_W) == 0, 1.0, 0.0)
    for h in range(2 * R):
        d = jnp.dot(degp_ref[h, 0] + degp_ref[h, 1], e0,
                    preferred_element_type=jnp.float32)
        out_ref[h] = lax.rsqrt(jnp.maximum(d, 1.0))


def _norm_call(degp):
    return pl.pallas_call(
        _norm_body,
        grid=(DEG_BINS // NORM_BLK,),
        in_specs=[pl.BlockSpec((2 * R, NC, NORM_BLK, DEG_W),
                               lambda i: (0, 0, i, 0))],
        out_specs=pl.BlockSpec((2 * R, NORM_BLK), lambda i: (0, i)),
        out_shape=jax.ShapeDtypeStruct((2 * R, DEG_BINS), jnp.float32),
    )(degp)


def _mm_body(x_ref, ns_ref, w_ref, out_ref):
    xb = x_ref[...] * ns_ref[0]
    out_ref[0] = jnp.dot(xb, w_ref[0], preferred_element_type=jnp.float32)


def _mm_call(x, ns, W):
    return pl.pallas_call(
        _mm_body,
        grid=(NB, R),
        in_specs=[
            pl.BlockSpec((128, D), lambda i, r: (i, 0)),
            pl.BlockSpec((1, 128, 1), lambda i, r: (r, i, 0)),
            pl.BlockSpec((1, D, D), lambda i, r: (r, 0, 0)),
        ],
        out_specs=pl.BlockSpec((1, 128, D), lambda i, r: (r, i, 0)),
        out_shape=jax.ShapeDtypeStruct((R, N, D), jnp.float32),
    )(x, ns, W)


def _comb_body(p0_ref, p1_ref, p2_ref, nd_ref, b_ref, out_ref, *, relu):
    acc = None
    for r, p in enumerate((p0_ref, p1_ref, p2_ref)):
        t = (p[0] + p[1]) * nd_ref[r] + b_ref[r][None, :]
        acc = t if acc is None else acc + t
    out_ref[...] = jnp.maximum(acc, 0.0) if relu else acc


def _comb_call(p0, p1, p2, nd, b, relu):
    return pl.pallas_call(
        functools.partial(_comb_body, relu=relu),
        grid=(NB,),
        in_specs=[
            pl.BlockSpec((NC, 128, D), lambda i: (0, i, 0)),
            pl.BlockSpec((NC, 128, D), lambda i: (0, i, 0)),
            pl.BlockSpec((NC, 128, D), lambda i: (0, i, 0)),
            pl.BlockSpec((R, 128, 1), lambda i: (0, i, 0)),
            pl.BlockSpec((R, D), lambda i: (0, 0)),
        ],
        out_specs=pl.BlockSpec((128, D), lambda i: (i, 0)),
        out_shape=jax.ShapeDtypeStruct((N, D), jnp.float32),
    )(p0, p1, p2, nd, b)


def kernel(x, edge_index_0, edge_index_1, edge_index_2,
           W1_0, b1_0, W2_0, b2_0,
           W1_1, b1_1, W2_1, b2_1,
           W1_2, b1_2, W2_2, b2_2):
    edges = [edge_index_0, edge_index_1, edge_index_2]
    npad = E_PAD - E
    pad_hi = jnp.full((npad,), N, jnp.int32)
    pad_lo = jnp.zeros((npad,), jnp.int32)

    deg_idx = [jnp.concatenate([e[i], pad_hi]) for e in edges for i in (0, 1)]
    src3 = [jnp.concatenate([e[0], pad_lo]) for e in edges]
    dst3 = [deg_idx[2 * r + 1] for r in range(R)]

    degp = _deg_call(deg_idx)
    norms = _norm_call(degp).reshape(R, 2, DEG_BINS)
    ns = norms[:, 0, :N, None]
    nd = norms[:, 1, :N, None]

    W1 = jnp.stack([W1_0, W1_1, W1_2])
    b1 = jnp.stack([b1_0, b1_1, b1_2])
    W2 = jnp.stack([W2_0, W2_1, W2_2])
    b2 = jnp.stack([b2_0, b2_1, b2_2])

    h1 = _mm_call(x, ns, W1)
    p0, p1, p2 = _seg_call(h1, src3, dst3)
    h = _comb_call(p0, p1, p2, nd, b1, relu=True)

    h2 = _mm_call(h, ns, W2)
    q0, q1, q2 = _seg_call(h2, src3, dst3)
    return _comb_call(q0, q1, q2, nd, b2, relu=False)


# speedup vs baseline: 2.0145x; 1.1980x over previous
import functools

import jax
import jax.numpy as jnp
from jax import lax
from jax.experimental import pallas as pl
from jax.experimental.pallas import tpu as pltpu
from jax.experimental.pallas import tpu_sc as plsc

N = 10000
E = 160000
D = 128
R = 3

NC = 2
NS = 16
NW = NC * NS

CHUNK = 128
E_PAD = 163840
EPW = E_PAD // NW
NCHUNK = EPW // CHUNK
ACC_ROWS = 10240
ROWS_PT = ACC_ROWS // NS
DEG_BINS = ACC_ROWS
NB = (N + 127) // 128

_mesh = lambda: plsc.VectorSubcoreMesh(core_axis_name="c", subcore_axis_name="s")


DEG_W = 128


def _deg_body(i0, i1, i2, i3, i4, i5, out_hbm,
              idx_v, ones_v, zeros_v, deg_sh, sem):
    c = lax.axis_index("c")
    s = lax.axis_index("s")
    z16 = jnp.zeros((16,), jnp.float32)
    o16 = jnp.ones((16,), jnp.float32)

    def ob(t, carry):
        ones_v[t // 8, pl.ds((t % 8) * 16, 16)] = o16
        return carry
    lax.fori_loop(0, CHUNK * (DEG_W // 16), ob, 0)

    def zb(t, carry):
        zeros_v[t // 8, pl.ds((t % 8) * 16, 16)] = z16
        return carry
    lax.fori_loop(0, CHUNK * (DEG_W // 16), zb, 0)

    wrow = (c * NS + s) * NCHUNK
    for h, idx_hbm in enumerate((i0, i1, i2, i3, i4, i5)):
        pltpu.sync_copy(idx_hbm.at[pl.ds(wrow, NCHUNK)], idx_v)
        zs = [pltpu.async_copy(
                  zeros_v,
                  deg_sh.at[pl.ds(s * ROWS_PT + k * CHUNK, CHUNK)], sem)
              for k in range(ROWS_PT // CHUNK)]
        for z in zs:
            z.wait()
        plsc.subcore_barrier()

        for batch in range(NCHUNK // 20):
            ds = [pltpu.async_copy(
                      ones_v, deg_sh.at[idx_v.at[batch * 20 + i]], sem,
                      add=True)
                  for i in range(20)]
            for d in ds:
                d.wait()
        plsc.subcore_barrier()

        pltpu.sync_copy(deg_sh.at[pl.ds(s * ROWS_PT, ROWS_PT)],
                        out_hbm.at[h, c, pl.ds(s * ROWS_PT, ROWS_PT)])
        plsc.subcore_barrier()


def _deg_call(deg_idx):
    return pl.kernel(
        _deg_body,
        out_type=jax.ShapeDtypeStruct((2 * R, NC, DEG_BINS, DEG_W), jnp.float32),
        mesh=_mesh(),
        scratch_types=[
            pltpu.VMEM((NCHUNK, CHUNK), jnp.int32),
            pltpu.VMEM((CHUNK, DEG_W), jnp.float32),
            pltpu.VMEM((CHUNK, DEG_W), jnp.float32),
            pltpu.VMEM_SHARED((DEG_BINS, DEG_W), jnp.float32),
            pltpu.SemaphoreType.DMA,
        ],
    )(*deg_idx)


NBUF = 2
NSS = NCHUNK // NBUF
ZROWS = 32


def _seg_body(h_hbm, s0, s1, s2, d0, d1, d2, o0, o1, o2,
              sv, dv, rows_v, zeros_v, acc,
              g0, g1, t0, t1, zsem):
    c = lax.axis_index("c")
    s = lax.axis_index("s")
    z16 = jnp.zeros((16,), jnp.float32)
    gsem = (g0, g1)
    ssem = (t0, t1)

    def zb(t, carry):
        zeros_v[t // 8, pl.ds((t % 8) * 16, 16)] = z16
        return carry
    lax.fori_loop(0, ZROWS * (D // 16), zb, 0)

    srcs = (s0, s1, s2)
    dsts = (d0, d1, d2)
    outs = (o0, o1, o2)
    wrow = (c * NS + s) * NCHUNK
    for r in range(R):
        pltpu.sync_copy(srcs[r].at[pl.ds(wrow, NCHUNK)], sv)
        pltpu.sync_copy(dsts[r].at[pl.ds(wrow, NCHUNK)], dv)
        zs = [pltpu.async_copy(
                  zeros_v, acc.at[pl.ds(s * ROWS_PT + k * ZROWS, ZROWS)],
                  zsem)
              for k in range(ROWS_PT // ZROWS)]
        for z in zs:
            z.wait()
        plsc.subcore_barrier()

        for b in range(NBUF):
            pltpu.async_copy(h_hbm.at[r].at[sv.at[b]], rows_v.at[b], gsem[b])

        def ss_body(ss, carry):
            base = ss * NBUF
            for b in range(NBUF):
                pltpu.make_async_copy(h_hbm.at[r].at[sv.at[base + b]],
                                      rows_v.at[b], gsem[b]).wait()
                pltpu.async_copy(rows_v.at[b], acc.at[dv.at[base + b]],
                                 ssem[b], add=True)
            for b in range(NBUF):
                @pl.when(ss < NSS - 1)
                def _prefetch():
                    pltpu.make_async_copy(rows_v.at[b],
                                          acc.at[dv.at[base + b]],
                                          ssem[b]).wait()
                    pltpu.async_copy(h_hbm.at[r].at[sv.at[base + NBUF + b]],
                                     rows_v.at[b], gsem[b])
            return carry
        lax.fori_loop(0, NSS, ss_body, 0)

        for b in range(NBUF):
            pltpu.make_async_copy(rows_v.at[b],
                                  acc.at[dv.at[(NSS - 1) * NBUF + b]],
                                  ssem[b]).wait()
        plsc.subcore_barrier()

        pltpu.sync_copy(acc.at[pl.ds(s * ROWS_PT, ROWS_PT)],
                        outs[r].at[c, pl.ds(s * ROWS_PT, ROWS_PT)])
        plsc.subcore_barrier()


def _seg_call(h3, src3, dst3):
    return pl.kernel(
        _seg_body,
        out_type=(jax.ShapeDtypeStruct((NC, ACC_ROWS, D), jnp.float32),) * R,
        mesh=_mesh(),
        scratch_types=[
            pltpu.VMEM((NCHUNK, CHUNK), jnp.int32),
            pltpu.VMEM((NCHUNK, CHUNK), jnp.int32),
            pltpu.VMEM((NBUF, CHUNK, D), jnp.float32),
            pltpu.VMEM((ZROWS, D), jnp.float32),
            pltpu.VMEM_SHARED((ACC_ROWS, D), jnp.float32),
        ] + [pltpu.SemaphoreType.DMA] * (2 * NBUF + 1),
    )(h3, *src3, *dst3)


NORM_BLK = 1024


def _norm_body(degp_ref, out_ref):
    e0 = jnp.where(lax.iota(jnp.int32, DEG_W) == 0, 1.0, 0.0)
    for h in range(2 * R):
        d = jnp.dot(degp_ref[h, 0] + degp_ref[h, 1], e0,
                    preferred_element_type=jnp.float32)
        out_ref[h] = lax.rsqrt(jnp.maximum(d, 1.0))


def _norm_call(degp):
    return pl.pallas_call(
        _norm_body,
        grid=(DEG_BINS // NORM_BLK,),
        in_specs=[pl.BlockSpec((2 * R, NC, NORM_BLK, DEG_W),
                               lambda i: (0, 0, i, 0))],
        out_specs=pl.BlockSpec((2 * R, NORM_BLK), lambda i: (0, i)),
        out_shape=jax.ShapeDtypeStruct((2 * R, DEG_BINS), jnp.float32),
    )(degp)


def _mm_body(x_ref, ns_ref, w_ref, out_ref):
    xb = x_ref[...] * ns_ref[0]
    out_ref[0] = jnp.dot(xb, w_ref[0], preferred_element_type=jnp.float32)


def _mm_call(x, ns, W):
    return pl.pallas_call(
        _mm_body,
        grid=(NB, R),
        in_specs=[
            pl.BlockSpec((128, D), lambda i, r: (i, 0)),
            pl.BlockSpec((1, 128, 1), lambda i, r: (r, i, 0)),
            pl.BlockSpec((1, D, D), lambda i, r: (r, 0, 0)),
        ],
        out_specs=pl.BlockSpec((1, 128, D), lambda i, r: (r, i, 0)),
        out_shape=jax.ShapeDtypeStruct((R, N, D), jnp.float32),
    )(x, ns, W)


def _comb_body(p0_ref, p1_ref, p2_ref, nd_ref, b_ref, out_ref, *, relu):
    acc = None
    for r, p in enumerate((p0_ref, p1_ref, p2_ref)):
        t = (p[0] + p[1]) * nd_ref[r] + b_ref[r][None, :]
        acc = t if acc is None else acc + t
    out_ref[...] = jnp.maximum(acc, 0.0) if relu else acc


def _comb_call(p0, p1, p2, nd, b, relu):
    return pl.pallas_call(
        functools.partial(_comb_body, relu=relu),
        grid=(NB,),
        in_specs=[
            pl.BlockSpec((NC, 128, D), lambda i: (0, i, 0)),
            pl.BlockSpec((NC, 128, D), lambda i: (0, i, 0)),
            pl.BlockSpec((NC, 128, D), lambda i: (0, i, 0)),
            pl.BlockSpec((R, 128, 1), lambda i: (0, i, 0)),
            pl.BlockSpec((R, D), lambda i: (0, 0)),
        ],
        out_specs=pl.BlockSpec((128, D), lambda i: (i, 0)),
        out_shape=jax.ShapeDtypeStruct((N, D), jnp.float32),
    )(p0, p1, p2, nd, b)


def kernel(x, edge_index_0, edge_index_1, edge_index_2,
           W1_0, b1_0, W2_0, b2_0,
           W1_1, b1_1, W2_1, b2_1,
           W1_2, b1_2, W2_2, b2_2):
    edges = [edge_index_0, edge_index_1, edge_index_2]
    npad = E_PAD - E
    pad_hi = jnp.full((npad,), N, jnp.int32)
    pad_lo = jnp.zeros((npad,), jnp.int32)

    blk = (E_PAD // CHUNK, CHUNK)
    deg_idx = [jnp.concatenate([e[i], pad_hi]).reshape(blk)
               for e in edges for i in (0, 1)]
    src3 = [jnp.concatenate([e[0], pad_lo]).reshape(blk) for e in edges]
    dst3 = [deg_idx[2 * r + 1] for r in range(R)]

    degp = _deg_call(deg_idx)
    norms = _norm_call(degp).reshape(R, 2, DEG_BINS)
    ns = norms[:, 0, :N, None]
    nd = norms[:, 1, :N, None]

    W1 = jnp.stack([W1_0, W1_1, W1_2])
    b1 = jnp.stack([b1_0, b1_1, b1_2])
    W2 = jnp.stack([W2_0, W2_1, W2_2])
    b2 = jnp.stack([b2_0, b2_1, b2_2])

    h1 = _mm_call(x, ns, W1)
    p0, p1, p2 = _seg_call(h1, src3, dst3)
    h = _comb_call(p0, p1, p2, nd, b1, relu=True)

    h2 = _mm_call(h, ns, W2)
    q0, q1, q2 = _seg_call(h2, src3, dst3)
    return _comb_call(q0, q1, q2, nd, b2, relu=False)


# speedup vs baseline: 2.0200x; 1.0028x over previous
import functools

import jax
import jax.numpy as jnp
from jax import lax
from jax.experimental import pallas as pl
from jax.experimental.pallas import tpu as pltpu
from jax.experimental.pallas import tpu_sc as plsc

N = 10000
E = 160000
D = 128
R = 3

NC = 2
NS = 16
NW = NC * NS

CHUNK = 128
E_PAD = 163840
EPW = E_PAD // NW
NCHUNK = EPW // CHUNK
ACC_ROWS = 10240
ROWS_PT = ACC_ROWS // NS
DEG_BINS = ACC_ROWS
NB = (N + 127) // 128

_mesh = lambda: plsc.VectorSubcoreMesh(core_axis_name="c", subcore_axis_name="s")


DEG_W = 128


def _deg_body(i0, i1, i2, i3, i4, i5, out_hbm,
              idx_v, ones_v, zeros_v, deg_sh, sem):
    c = lax.axis_index("c")
    s = lax.axis_index("s")
    z16 = jnp.zeros((16,), jnp.float32)
    o16 = jnp.ones((16,), jnp.float32)

    def ob(t, carry):
        ones_v[t // 8, pl.ds((t % 8) * 16, 16)] = o16
        return carry
    lax.fori_loop(0, CHUNK * (DEG_W // 16), ob, 0)

    def zb(t, carry):
        zeros_v[t // 8, pl.ds((t % 8) * 16, 16)] = z16
        return carry
    lax.fori_loop(0, CHUNK * (DEG_W // 16), zb, 0)

    wrow = (c * NS + s) * NCHUNK
    for h, idx_hbm in enumerate((i0, i1, i2, i3, i4, i5)):
        pltpu.sync_copy(idx_hbm.at[pl.ds(wrow, NCHUNK)], idx_v)
        zs = [pltpu.async_copy(
                  zeros_v,
                  deg_sh.at[pl.ds(s * ROWS_PT + k * CHUNK, CHUNK)], sem)
              for k in range(ROWS_PT // CHUNK)]
        for z in zs:
            z.wait()
        plsc.subcore_barrier()

        for batch in range(NCHUNK // 20):
            ds = [pltpu.async_copy(
                      ones_v, deg_sh.at[idx_v.at[batch * 20 + i]], sem,
                      add=True)
                  for i in range(20)]
            for d in ds:
                d.wait()
        plsc.subcore_barrier()

        pltpu.sync_copy(deg_sh.at[pl.ds(s * ROWS_PT, ROWS_PT)],
                        out_hbm.at[h, c, pl.ds(s * ROWS_PT, ROWS_PT)])
        plsc.subcore_barrier()


def _deg_call(deg_idx):
    return pl.kernel(
        _deg_body,
        out_type=jax.ShapeDtypeStruct((2 * R, NC, DEG_BINS, DEG_W), jnp.float32),
        mesh=_mesh(),
        scratch_types=[
            pltpu.VMEM((NCHUNK, CHUNK), jnp.int32),
            pltpu.VMEM((CHUNK, DEG_W), jnp.float32),
            pltpu.VMEM((CHUNK, DEG_W), jnp.float32),
            pltpu.VMEM_SHARED((DEG_BINS, DEG_W), jnp.float32),
            pltpu.SemaphoreType.DMA,
        ],
    )(*deg_idx)


NBUF = 2
NSS = NCHUNK // NBUF
ZROWS = 32


def _seg_body(h_hbm, s0, s1, s2, d0, d1, d2, o0, o1, o2,
              sv, dv, rows_v, zeros_v, acc,
              g0, g1, t0, t1, zsem):
    c = lax.axis_index("c")
    s = lax.axis_index("s")
    z16 = jnp.zeros((16,), jnp.float32)
    gsem = (g0, g1)
    ssem = (t0, t1)

    def zb(t, carry):
        zeros_v[t // 8, pl.ds((t % 8) * 16, 16)] = z16
        return carry
    lax.fori_loop(0, ZROWS * (D // 16), zb, 0)

    srcs = (s0, s1, s2)
    dsts = (d0, d1, d2)
    outs = (o0, o1, o2)
    wrow = (c * NS + s) * NCHUNK
    for r in range(R):
        pltpu.sync_copy(srcs[r].at[pl.ds(wrow, NCHUNK)], sv)
        pltpu.sync_copy(dsts[r].at[pl.ds(wrow, NCHUNK)], dv)
        zs = [pltpu.async_copy(
                  zeros_v, acc.at[pl.ds(s * ROWS_PT + k * ZROWS, ZROWS)],
                  zsem)
              for k in range(ROWS_PT // ZROWS)]
        for z in zs:
            z.wait()
        plsc.subcore_barrier()

        for b in range(NBUF):
            pltpu.async_copy(h_hbm.at[r].at[sv.at[b]], rows_v.at[b], gsem[b])

        def ss_body(ss, carry):
            base = ss * NBUF
            for b in range(NBUF):
                pltpu.make_async_copy(h_hbm.at[r].at[sv.at[base + b]],
                                      rows_v.at[b], gsem[b]).wait()
                pltpu.async_copy(rows_v.at[b], acc.at[dv.at[base + b]],
                                 ssem[b], add=True)
            for b in range(NBUF):
                @pl.when(ss < NSS - 1)
                def _prefetch():
                    pltpu.make_async_copy(rows_v.at[b],
                                          acc.at[dv.at[base + b]],
                                          ssem[b]).wait()
                    pltpu.async_copy(h_hbm.at[r].at[sv.at[base + NBUF + b]],
                                     rows_v.at[b], gsem[b])
            return carry
        lax.fori_loop(0, NSS, ss_body, 0)

        for b in range(NBUF):
            pltpu.make_async_copy(rows_v.at[b],
                                  acc.at[dv.at[(NSS - 1) * NBUF + b]],
                                  ssem[b]).wait()
        plsc.subcore_barrier()

        pltpu.sync_copy(acc.at[pl.ds(s * ROWS_PT, ROWS_PT)],
                        outs[r].at[c, pl.ds(s * ROWS_PT, ROWS_PT)])
        plsc.subcore_barrier()


def _seg_call(h3, src3, dst3):
    return pl.kernel(
        _seg_body,
        out_type=(jax.ShapeDtypeStruct((NC, ACC_ROWS, D), jnp.float32),) * R,
        mesh=_mesh(),
        scratch_types=[
            pltpu.VMEM((NCHUNK, CHUNK), jnp.int32),
            pltpu.VMEM((NCHUNK, CHUNK), jnp.int32),
            pltpu.VMEM((NBUF, CHUNK, D), jnp.float32),
            pltpu.VMEM((ZROWS, D), jnp.float32),
            pltpu.VMEM_SHARED((ACC_ROWS, D), jnp.float32),
        ] + [pltpu.SemaphoreType.DMA] * (2 * NBUF + 1),
    )(h3, *src3, *dst3)


NORM_BLK = 1024


def _norm_body(degp_ref, out_ref):
    e0 = jnp.where(lax.iota(jnp.int32, DEG_W) == 0, 1.0, 0.0)
    for h in range(2 * R):
        d = jnp.dot(degp_ref[h, 0] + degp_ref[h, 1], e0,
                    preferred_element_type=jnp.float32)
        out_ref[h] = lax.rsqrt(jnp.maximum(d, 1.0))


def _norm_call(degp):
    return pl.pallas_call(
        _norm_body,
        grid=(DEG_BINS // NORM_BLK,),
        in_specs=[pl.BlockSpec((2 * R, NC, NORM_BLK, DEG_W),
                               lambda i: (0, 0, i, 0))],
        out_specs=pl.BlockSpec((2 * R, NORM_BLK), lambda i: (0, i)),
        out_shape=jax.ShapeDtypeStruct((2 * R, DEG_BINS), jnp.float32),
    )(degp)


def _mm_body(x_ref, ns_ref, w_ref, out_ref):
    xb = x_ref[...] * ns_ref[0]
    out_ref[0] = jnp.dot(xb, w_ref[0], preferred_element_type=jnp.float32)


def _mm_call(x, ns, W):
    return pl.pallas_call(
        _mm_body,
        grid=(NB, R),
        in_specs=[
            pl.BlockSpec((128, D), lambda i, r: (i, 0)),
            pl.BlockSpec((1, 128, 1), lambda i, r: (r, i, 0)),
            pl.BlockSpec((1, D, D), lambda i, r: (r, 0, 0)),
        ],
        out_specs=pl.BlockSpec((1, 128, D), lambda i, r: (r, i, 0)),
        out_shape=jax.ShapeDtypeStruct((R, N, D), jnp.float32),
    )(x, ns, W)


def _comb_body(p0_ref, p1_ref, p2_ref, nd_ref, b_ref, out_ref, *, relu):
    acc = None
    for r, p in enumerate((p0_ref, p1_ref, p2_ref)):
        t = (p[0] + p[1]) * nd_ref[r] + b_ref[r][None, :]
        acc = t if acc is None else acc + t
    out_ref[...] = jnp.maximum(acc, 0.0) if relu else acc


def _comb_call(p0, p1, p2, nd, b, relu):
    return pl.pallas_call(
        functools.partial(_comb_body, relu=relu),
        grid=(NB,),
        in_specs=[
            pl.BlockSpec((NC, 128, D), lambda i: (0, i, 0)),
            pl.BlockSpec((NC, 128, D), lambda i: (0, i, 0)),
            pl.BlockSpec((NC, 128, D), lambda i: (0, i, 0)),
            pl.BlockSpec((R, 128, 1), lambda i: (0, i, 0)),
            pl.BlockSpec((R, D), lambda i: (0, 0)),
        ],
        out_specs=pl.BlockSpec((128, D), lambda i: (i, 0)),
        out_shape=jax.ShapeDtypeStruct((N, D), jnp.float32),
    )(p0, p1, p2, nd, b)


def kernel(x, edge_index_0, edge_index_1, edge_index_2,
           W1_0, b1_0, W2_0, b2_0,
           W1_1, b1_1, W2_1, b2_1,
           W1_2, b1_2, W2_2, b2_2):
    edges = [edge_index_0, edge_index_1, edge_index_2]
    npad = E_PAD - E
    pad_hi = N + jnp.arange(npad, dtype=jnp.int32) % (ACC_ROWS - N)
    pad_lo = jnp.zeros((npad,), jnp.int32)

    blk = (E_PAD // CHUNK, CHUNK)
    deg_idx = [jnp.concatenate([e[i], pad_hi]).reshape(blk)
               for e in edges for i in (0, 1)]
    src3 = [jnp.concatenate([e[0], pad_lo]).reshape(blk) for e in edges]
    dst3 = [deg_idx[2 * r + 1] for r in range(R)]

    degp = _deg_call(deg_idx)
    norms = _norm_call(degp).reshape(R, 2, DEG_BINS)
    ns = norms[:, 0, :N, None]
    nd = norms[:, 1, :N, None]

    W1 = jnp.stack([W1_0, W1_1, W1_2])
    b1 = jnp.stack([b1_0, b1_1, b1_2])
    W2 = jnp.stack([W2_0, W2_1, W2_2])
    b2 = jnp.stack([b2_0, b2_1, b2_2])

    h1 = _mm_call(x, ns, W1)
    p0, p1, p2 = _seg_call(h1, src3, dst3)
    h = _comb_call(p0, p1, p2, nd, b1, relu=True)

    h2 = _mm_call(h, ns, W2)
    q0, q1, q2 = _seg_call(h2, src3, dst3)
    return _comb_call(q0, q1, q2, nd, b2, relu=False)


# speedup vs baseline: 2.1056x; 1.0424x over previous
import functools

import jax
import jax.numpy as jnp
from jax import lax
from jax.experimental import pallas as pl
from jax.experimental.pallas import tpu as pltpu
from jax.experimental.pallas import tpu_sc as plsc

N = 10000
E = 160000
D = 128
R = 3

NC = 2
NS = 16
NW = NC * NS

CHUNK = 128
E_PAD = 163840
NROWS = E_PAD // CHUNK
ACC_ROWS = 10112
ROWS_PT = ACC_ROWS // NS
DEG_BINS = 10240
DROWS_PT = DEG_BINS // NS
NB = (N + 127) // 128

_mesh = lambda: plsc.VectorSubcoreMesh(core_axis_name="c", subcore_axis_name="s")


DEG_W = 128
DEG_NCH = NROWS // NW


def _deg_body(i0, i1, i2, i3, i4, i5, out_hbm,
              idx_v, ones_v, zeros_v, deg_sh, sem):
    c = lax.axis_index("c")
    s = lax.axis_index("s")
    z16 = jnp.zeros((16,), jnp.float32)
    o16 = jnp.ones((16,), jnp.float32)

    def ob(t, carry):
        ones_v[t // 8, pl.ds((t % 8) * 16, 16)] = o16
        return carry
    lax.fori_loop(0, CHUNK * (DEG_W // 16), ob, 0)

    def zb(t, carry):
        zeros_v[t // 8, pl.ds((t % 8) * 16, 16)] = z16
        return carry
    lax.fori_loop(0, CHUNK * (DEG_W // 16), zb, 0)

    wrow = (c * NS + s) * DEG_NCH
    for h, idx_hbm in enumerate((i0, i1, i2, i3, i4, i5)):
        pltpu.sync_copy(idx_hbm.at[pl.ds(wrow, DEG_NCH)], idx_v)
        zs = [pltpu.async_copy(
                  zeros_v,
                  deg_sh.at[pl.ds(s * DROWS_PT + k * CHUNK, CHUNK)], sem)
              for k in range(DROWS_PT // CHUNK)]
        for z in zs:
            z.wait()
        plsc.subcore_barrier()

        for batch in range(DEG_NCH // 20):
            ds = [pltpu.async_copy(
                      ones_v, deg_sh.at[idx_v.at[batch * 20 + i]], sem,
                      add=True)
                  for i in range(20)]
            for d in ds:
                d.wait()
        plsc.subcore_barrier()

        pltpu.sync_copy(deg_sh.at[pl.ds(s * DROWS_PT, DROWS_PT)],
                        out_hbm.at[h, c, pl.ds(s * DROWS_PT, DROWS_PT)])
        plsc.subcore_barrier()


def _deg_call(deg_idx):
    return pl.kernel(
        _deg_body,
        out_type=jax.ShapeDtypeStruct((2 * R, NC, DEG_BINS, DEG_W), jnp.float32),
        mesh=_mesh(),
        scratch_types=[
            pltpu.VMEM((DEG_NCH, CHUNK), jnp.int32),
            pltpu.VMEM((CHUNK, DEG_W), jnp.float32),
            pltpu.VMEM((CHUNK, DEG_W), jnp.float32),
            pltpu.VMEM_SHARED((DEG_BINS, DEG_W), jnp.float32),
            pltpu.SemaphoreType.DMA,
        ],
    )(*deg_idx)


NBUF = 2
NCH0 = 56
NCH1 = 24
NSS0 = NCH0 // NBUF
NSS1 = NCH1 // NBUF
ZROWS = 16


def _seg_body(h_hbm, s0, s1, s2, d0, d1, d2, o0, o1, o2,
              sv, dv, rows_v, zeros_v, acc,
              g0, g1, t0, t1, zsem):
    c = lax.axis_index("c")
    s = lax.axis_index("s")
    z16 = jnp.zeros((16,), jnp.float32)
    gsem = (g0, g1)
    ssem = (t0, t1)

    ncw = jnp.where(c == 0, NCH0, NCH1)
    nss = jnp.where(c == 0, NSS0, NSS1)
    row0 = c * (NS * NCH0) + s * ncw

    def zb(t, carry):
        zeros_v[t // 8, pl.ds((t % 8) * 16, 16)] = z16
        return carry
    lax.fori_loop(0, ZROWS * (D // 16), zb, 0)

    srcs = (s0, s1, s2)
    dsts = (d0, d1, d2)
    outs = (o0, o1, o2)
    for r in range(R):
        pltpu.sync_copy(srcs[r].at[pl.ds(row0, NCH1)], sv.at[pl.ds(0, NCH1)])
        pltpu.sync_copy(dsts[r].at[pl.ds(row0, NCH1)], dv.at[pl.ds(0, NCH1)])

        @pl.when(c == 0)
        def _rest():
            pltpu.sync_copy(srcs[r].at[pl.ds(row0 + NCH1, NCH0 - NCH1)],
                            sv.at[pl.ds(NCH1, NCH0 - NCH1)])
            pltpu.sync_copy(dsts[r].at[pl.ds(row0 + NCH1, NCH0 - NCH1)],
                            dv.at[pl.ds(NCH1, NCH0 - NCH1)])

        zs = [pltpu.async_copy(
                  zeros_v, acc.at[pl.ds(s * ROWS_PT + k * ZROWS, ZROWS)],
                  zsem)
              for k in range(ROWS_PT // ZROWS)]
        zs.append(pltpu.async_copy(
            zeros_v.at[pl.ds(0, 8)],
            acc.at[pl.ds(s * ROWS_PT + (ROWS_PT // ZROWS) * ZROWS, 8)], zsem))
        for z in zs:
            z.wait()
        plsc.subcore_barrier()

        for b in range(NBUF):
            pltpu.async_copy(h_hbm.at[r].at[sv.at[b]], rows_v.at[b], gsem[b])

        def ss_body(ss, carry):
            base = ss * NBUF
            for b in range(NBUF):
                pltpu.make_async_copy(h_hbm.at[r].at[sv.at[base + b]],
                                      rows_v.at[b], gsem[b]).wait()
                pltpu.async_copy(rows_v.at[b], acc.at[dv.at[base + b]],
                                 ssem[b], add=True)
            for b in range(NBUF):
                @pl.when(ss < nss - 1)
                def _prefetch():
                    pltpu.make_async_copy(rows_v.at[b],
                                          acc.at[dv.at[base + b]],
                                          ssem[b]).wait()
                    pltpu.async_copy(h_hbm.at[r].at[sv.at[base + NBUF + b]],
                                     rows_v.at[b], gsem[b])
            return carry
        lax.fori_loop(0, nss, ss_body, 0)

        for b in range(NBUF):
            pltpu.make_async_copy(rows_v.at[b],
                                  acc.at[dv.at[(nss - 1) * NBUF + b]],
                                  ssem[b]).wait()
        plsc.subcore_barrier()

        pltpu.sync_copy(acc.at[pl.ds(s * ROWS_PT, ROWS_PT)],
                        outs[r].at[c, pl.ds(s * ROWS_PT, ROWS_PT)])
        plsc.subcore_barrier()


def _seg_call(h3, src3, dst3):
    return pl.kernel(
        _seg_body,
        out_type=(jax.ShapeDtypeStruct((NC, ACC_ROWS, D), jnp.float32),) * R,
        mesh=_mesh(),
        scratch_types=[
            pltpu.VMEM((NCH0, CHUNK), jnp.int32),
            pltpu.VMEM((NCH0, CHUNK), jnp.int32),
            pltpu.VMEM((NBUF, CHUNK, D), jnp.float32),
            pltpu.VMEM((ZROWS, D), jnp.float32),
            pltpu.VMEM_SHARED((ACC_ROWS, D), jnp.float32),
        ] + [pltpu.SemaphoreType.DMA] * (2 * NBUF + 1),
    )(h3, *src3, *dst3)


NORM_BLK = 1024


def _norm_body(degp_ref, out_ref):
    e0 = jnp.where(lax.iota(jnp.int32, DEG_W) == 0, 1.0, 0.0)
    for h in range(2 * R):
        d = jnp.dot(degp_ref[h, 0] + degp_ref[h, 1], e0,
                    preferred_element_type=jnp.float32)
        out_ref[h] = lax.rsqrt(jnp.maximum(d, 1.0))


def _norm_call(degp):
    return pl.pallas_call(
        _norm_body,
        grid=(DEG_BINS // NORM_BLK,),
        in_specs=[pl.BlockSpec((2 * R, NC, NORM_BLK, DEG_W),
                               lambda i: (0, 0, i, 0))],
        out_specs=pl.BlockSpec((2 * R, NORM_BLK), lambda i: (0, i)),
        out_shape=jax.ShapeDtypeStruct((2 * R, DEG_BINS), jnp.float32),
    )(degp)


def _mm_body(x_ref, ns_ref, w_ref, out_ref):
    xb = x_ref[...] * ns_ref[0]
    out_ref[0] = jnp.dot(xb, w_ref[0], preferred_element_type=jnp.float32)


def _mm_call(x, ns, W):
    return pl.pallas_call(
        _mm_body,
        grid=(NB, R),
        in_specs=[
            pl.BlockSpec((128, D), lambda i, r: (i, 0)),
            pl.BlockSpec((1, 128, 1), lambda i, r: (r, i, 0)),
            pl.BlockSpec((1, D, D), lambda i, r: (r, 0, 0)),
        ],
        out_specs=pl.BlockSpec((1, 128, D), lambda i, r: (r, i, 0)),
        out_shape=jax.ShapeDtypeStruct((R, N, D), jnp.float32),
    )(x, ns, W)


def _comb_body(p0_ref, p1_ref, p2_ref, nd_ref, b_ref, out_ref, *, relu):
    acc = None
    for r, p in enumerate((p0_ref, p1_ref, p2_ref)):
        t = (p[0] + p[1]) * nd_ref[r] + b_ref[r][None, :]
        acc = t if acc is None else acc + t
    out_ref[...] = jnp.maximum(acc, 0.0) if relu else acc


def _comb_call(p0, p1, p2, nd, b, relu):
    return pl.pallas_call(
        functools.partial(_comb_body, relu=relu),
        grid=(NB,),
        in_specs=[
            pl.BlockSpec((NC, 128, D), lambda i: (0, i, 0)),
            pl.BlockSpec((NC, 128, D), lambda i: (0, i, 0)),
            pl.BlockSpec((NC, 128, D), lambda i: (0, i, 0)),
            pl.BlockSpec((R, 128, 1), lambda i: (0, i, 0)),
            pl.BlockSpec((R, D), lambda i: (0, 0)),
        ],
        out_specs=pl.BlockSpec((128, D), lambda i: (i, 0)),
        out_shape=jax.ShapeDtypeStruct((N, D), jnp.float32),
    )(p0, p1, p2, nd, b)


def kernel(x, edge_index_0, edge_index_1, edge_index_2,
           W1_0, b1_0, W2_0, b2_0,
           W1_1, b1_1, W2_1, b2_1,
           W1_2, b1_2, W2_2, b2_2):
    edges = [edge_index_0, edge_index_1, edge_index_2]
    npad = E_PAD - E
    pad_hi = N + jnp.arange(npad, dtype=jnp.int32) % (ACC_ROWS - N)
    pad_lo = jnp.zeros((npad,), jnp.int32)

    blk = (NROWS, CHUNK)
    deg_idx = [jnp.concatenate([e[i], pad_hi]).reshape(blk)
               for e in edges for i in (0, 1)]
    src3 = [jnp.concatenate([e[0], pad_lo]).reshape(blk) for e in edges]
    dst3 = [deg_idx[2 * r + 1] for r in range(R)]

    degp = _deg_call(deg_idx)
    norms = _norm_call(degp).reshape(R, 2, DEG_BINS)
    ns = norms[:, 0, :N, None]
    nd = norms[:, 1, :N, None]

    W1 = jnp.stack([W1_0, W1_1, W1_2])
    b1 = jnp.stack([b1_0, b1_1, b1_2])
    W2 = jnp.stack([W2_0, W2_1, W2_2])
    b2 = jnp.stack([b2_0, b2_1, b2_2])

    h1 = _mm_call(x, ns, W1)
    p0, p1, p2 = _seg_call(h1, src3, dst3)
    h = _comb_call(p0, p1, p2, nd, b1, relu=True)

    h2 = _mm_call(h, ns, W2)
    q0, q1, q2 = _seg_call(h2, src3, dst3)
    return _comb_call(q0, q1, q2, nd, b2, relu=False)
